# baseline (speedup 1.0000x reference)
import functools

import jax
import jax.numpy as jnp
from jax import lax
from jax.experimental import pallas as pl
from jax.experimental.pallas import tpu as pltpu
from jax.experimental.pallas import tpu_sc as plsc

_VOCAB = 1000
_VOCAB_PAD = 1024
_MIN_OCC = 2
_NC = 2
_NS = 16
_NW = _NC * _NS
_L = 16
_GRP = 128
_W = 128
_HR = 16


def _tc_prepare(xv5):
    b, ndhi, nthi, _, _ = xv5.shape
    d = ndhi * 8
    tj = 4

    def body(x_ref, o_ref):
        eye = (lax.broadcasted_iota(jnp.int32, (_W, _W), 0)
               == lax.broadcasted_iota(jnp.int32, (_W, _W), 1)
               ).astype(jnp.bfloat16)
        dn = (((0,), (0,)), ((), ()))
        for i in range(tj):
            xb = x_ref[0, :, i, :, :].reshape(d, _GRP)
            xb2 = jnp.concatenate([xb, xb * xb], axis=0)
            hi = xb2.astype(jnp.bfloat16)
            lo = (xb2 - hi.astype(jnp.float32)).astype(jnp.bfloat16)
            xt = (lax.dot_general(hi, eye, dn,
                                  preferred_element_type=jnp.float32)
                  + lax.dot_general(lo, eye, dn,
                                    preferred_element_type=jnp.float32))
            o_ref[pl.ds(i * _GRP, _GRP), :] = xt

    return pl.pallas_call(
        body,
        grid=(b, nthi // tj),
        in_specs=[pl.BlockSpec((1, ndhi, tj, 8, _GRP),
                               lambda i, j: (i, 0, j, 0, 0))],
        out_specs=pl.BlockSpec((tj * _GRP, _W),
                               lambda i, j: (i * (nthi // tj) + j, 0)),
        out_shape=jax.ShapeDtypeStruct((b * nthi * _GRP, _W), jnp.float32),
    )(xv5)


def _sc_segment_stats(comb_rows, tv):
    nthi, nb, _ = tv.shape
    ngrp = (nthi * nb) // _NW
    tq_per_b = nthi // ngrp
    chunk = ngrp * _GRP
    rows_w = _VOCAB_PAD // _NS

    mesh = plsc.VectorSubcoreMesh(core_axis_name="c", subcore_axis_name="s")

    @functools.partial(
        pl.kernel,
        out_type=(
            jax.ShapeDtypeStruct((_NC * _VOCAB_PAD, _W), jnp.float32),
            jax.ShapeDtypeStruct((_NC * _HR, _W), jnp.float32),
        ),
        mesh=mesh,
        compiler_params=pltpu.CompilerParams(
            use_tc_tiling_on_sc=False, needs_layout_passes=False),
        scratch_types=(
            pltpu.VMEM((ngrp, _GRP), jnp.int32),
            pltpu.VMEM((chunk, _W), jnp.float32),
            pltpu.VMEM((rows_w, _W), jnp.float32),
            pltpu.VMEM((_HR, _W), jnp.float32),
            pltpu.VMEM((_L,), jnp.int32),
            pltpu.VMEM_SHARED((_VOCAB_PAD, _W), jnp.float32),
            pltpu.VMEM_SHARED((_HR, _W), jnp.float32),
            pltpu.SemaphoreType.DMA,
            pltpu.SemaphoreType.DMA,
        ),
    )
    def k(x_hbm, t_hbm, acc_hbm, cnt_hbm,
          idx_v, comb, z_v, hist, iota_v, acc, acc_c, in_sem, add_sem):
        cid = lax.axis_index("c")
        sid = lax.axis_index("s")
        wid = cid * _NS + sid
        b = wid // tq_per_b
        tq = wid % tq_per_b

        in_t = [pltpu.async_copy(t_hbm.at[tq * ngrp + g, b, :],
                                 idx_v.at[g], add_sem)
                for g in range(ngrp)]
        in_x = pltpu.async_copy(
            x_hbm.at[pl.ds((b * nthi + tq * ngrp) * _GRP, chunk), :],
            comb, in_sem)

        zero = jnp.zeros((_L,), jnp.float32)
        one = jnp.ones((_L,), jnp.float32)
        iota_v[...] = lax.iota(jnp.int32, _L)

        @pl.loop(0, rows_w)
        def _(r):
            @pl.loop(0, _W, step=_L)
            def _(c0):
                z_v[r, pl.ds(c0, _L)] = zero

        @pl.loop(0, _HR)
        def _(r):
            @pl.loop(0, _W, step=_L)
            def _(c0):
                hist[r, pl.ds(c0, _L)] = zero

        vbase = sid * rows_w
        z0 = pltpu.async_copy(z_v, acc.at[pl.ds(vbase, rows_w), :], add_sem)

        @pl.when(sid == 0)
        def _():
            pltpu.async_copy(z_v.at[pl.ds(0, _HR), :], acc_c, add_sem).wait()

        for c in in_t:
            c.wait()
        z0.wait()

        @pl.loop(0, ngrp)
        def _(g):
            @pl.loop(0, _GRP, step=_L)
            def _(j):
                ids = idx_v[g, pl.ds(j, _L)]
                plsc.addupdate_scatter(
                    hist,
                    [lax.shift_right_logical(ids, 7),
                     lax.bitwise_and(ids, 127)],
                    one)

        in_x.wait()
        plsc.subcore_barrier()

        adds = []
        for g in range(ngrp):
            adds.append(pltpu.async_copy(comb.at[pl.ds(g * _GRP, _GRP), :],
                                         acc.at[idx_v.at[g]], add_sem,
                                         add=True))
        adds.append(pltpu.async_copy(hist, acc_c.at[iota_v], add_sem,
                                     add=True))
        for a in adds:
            a.wait()

        plsc.subcore_barrier()

        obase = cid * _VOCAB_PAD + vbase
        w0 = pltpu.async_copy(acc.at[pl.ds(vbase, rows_w), :],
                              acc_hbm.at[pl.ds(obase, rows_w), :], add_sem)

        @pl.when(sid == 0)
        def _():
            pltpu.async_copy(acc_c,
                             cnt_hbm.at[pl.ds(cid * _HR, _HR), :],
                             add_sem).wait()

        w0.wait()

    return k(comb_rows, tv)


def _finalize_tc(acc, counts):
    vp = _VOCAB_PAD

    def body(a_ref, c_ref, loss_ref, nrep_ref):
        s = a_ref[:vp, 0:64] + a_ref[vp:, 0:64]
        q = a_ref[:vp, 64:128] + a_ref[vp:, 64:128]
        c8 = c_ref[0:8, :] + c_ref[_HR:_HR + 8, :]
        i_row = lax.broadcasted_iota(jnp.int32, (vp, 1), 0)
        hi = lax.shift_right_logical(i_row, 7)
        lo = lax.bitwise_and(i_row, 127)
        lane = lax.broadcasted_iota(jnp.int32, (vp, _W), 1)
        sel = jnp.zeros((vp, _W), jnp.float32)
        for r in range(8):
            sel = jnp.where(hi == r, c8[r:r + 1, :], sel)
        c = jnp.sum(jnp.where(lane == lo, sel, 0.0), axis=1, keepdims=True)
        mean = s / jnp.maximum(c, 1.0)
        ss = q - c * mean * mean
        var = ss / jnp.maximum(c - 1.0, 1.0)
        var_mean = jnp.sum(var, axis=1, keepdims=True) / var.shape[1]
        repeated = c >= float(_MIN_OCC)
        nrep = jnp.sum(repeated.astype(jnp.int32))
        total = jnp.sum(jnp.where(repeated, var_mean, 0.0))
        avg = total / jnp.maximum(nrep.astype(jnp.float32), 1.0)
        loss = jnp.clip(1.0 - avg, 0.0, None)
        loss_ref[0, 0] = jnp.where(nrep > 0, loss, jnp.float32(0.0))
        nrep_ref[0, 0] = nrep

    return pl.pallas_call(
        body,
        out_shape=(
            jax.ShapeDtypeStruct((1, 1), jnp.float32),
            jax.ShapeDtypeStruct((1, 1), jnp.int32),
        ),
        out_specs=(
            pl.BlockSpec(memory_space=pltpu.SMEM),
            pl.BlockSpec(memory_space=pltpu.SMEM),
        ),
    )(acc, counts)


@jax.jit
def kernel(semantic_state, token_ids):
    b, t_len, d = semantic_state.shape
    nthi = t_len // _GRP
    xv5 = semantic_state.reshape(b, nthi, _GRP, d // 8, 8)
    xv5 = xv5.transpose(0, 3, 1, 4, 2)
    comb_rows = _tc_prepare(xv5)
    tv = token_ids.astype(jnp.int32).reshape(b, nthi, _GRP).transpose(1, 0, 2)
    acc, counts = _sc_segment_stats(comb_rows, tv)
    loss, nrep = _finalize_tc(acc, counts)
    return loss[0, 0], nrep[0, 0]


# speedup vs baseline: 1.1346x; 1.1346x over previous
import functools

import jax
import jax.numpy as jnp
from jax import lax
from jax.experimental import pallas as pl
from jax.experimental.pallas import tpu as pltpu
from jax.experimental.pallas import tpu_sc as plsc

_VOCAB = 1000
_VOCAB_PAD = 1024
_MIN_OCC = 2
_NC = 2
_NS = 16
_NW = _NC * _NS
_L = 16
_GRP = 128
_W = 128
_HR = 16


def _sc_segment_stats(x, t):
    n, d = x.shape
    nthi, nb, _ = t.shape
    chunk = n // _NW
    ngrp = chunk // _GRP
    tq_per_b = nthi // ngrp
    rows_w = _VOCAB_PAD // _NS

    mesh = plsc.VectorSubcoreMesh(core_axis_name="c", subcore_axis_name="s")

    @functools.partial(
        pl.kernel,
        out_type=(
            jax.ShapeDtypeStruct((_NC * _VOCAB_PAD, _W), jnp.float32),
            jax.ShapeDtypeStruct((_NC * _HR, _W), jnp.float32),
        ),
        mesh=mesh,
        compiler_params=pltpu.CompilerParams(
            use_tc_tiling_on_sc=False, needs_layout_passes=False),
        scratch_types=(
            pltpu.VMEM((ngrp, _GRP), jnp.int32),
            pltpu.VMEM((chunk, _W), jnp.float32),
            pltpu.VMEM((rows_w, _W), jnp.float32),
            pltpu.VMEM((_HR, _W), jnp.float32),
            pltpu.VMEM((_L,), jnp.int32),
            pltpu.VMEM_SHARED((_VOCAB_PAD, _W), jnp.float32),
            pltpu.VMEM_SHARED((_HR, _W), jnp.float32),
            pltpu.SemaphoreType.DMA,
            pltpu.SemaphoreType.DMA,
        ),
    )
    def k(x_hbm, t_hbm, acc_hbm, cnt_hbm,
          idx_v, comb, z_v, hist, iota_v, acc, acc_c, in_sem, add_sem):
        cid = lax.axis_index("c")
        sid = lax.axis_index("s")
        wid = cid * _NS + sid

        b = wid // tq_per_b
        tq = wid % tq_per_b

        in_t = [pltpu.async_copy(t_hbm.at[tq * ngrp + g, b, :],
                                 idx_v.at[g], add_sem)
                for g in range(ngrp)]
        in_x = pltpu.async_copy(x_hbm.at[pl.ds(wid * chunk, chunk), :],
                                comb.at[:, pl.ds(0, d)], in_sem)

        zero = jnp.zeros((_L,), jnp.float32)
        one = jnp.ones((_L,), jnp.float32)
        iota_v[...] = lax.iota(jnp.int32, _L)

        @pl.loop(0, rows_w)
        def _(r):
            @pl.loop(0, _W, step=_L)
            def _(c0):
                z_v[r, pl.ds(c0, _L)] = zero

        @pl.loop(0, _HR)
        def _(r):
            @pl.loop(0, _W, step=_L)
            def _(c0):
                hist[r, pl.ds(c0, _L)] = zero

        vbase = sid * rows_w
        z0 = pltpu.async_copy(z_v, acc.at[pl.ds(vbase, rows_w), :], add_sem)

        @pl.when(sid == 0)
        def _():
            pltpu.async_copy(z_v.at[pl.ds(0, _HR), :], acc_c, add_sem).wait()

        for cpy in in_t:
            cpy.wait()
        z0.wait()

        @pl.loop(0, ngrp)
        def _(g):
            @pl.loop(0, _GRP, step=_L)
            def _(j):
                ids = idx_v[g, pl.ds(j, _L)]
                r = lax.shift_right_logical(ids, 7)
                c = lax.bitwise_and(ids, 127)
                plsc.addupdate_scatter(hist, [r, c], one)

        in_x.wait()

        @pl.loop(0, chunk)
        def _(r):
            @pl.loop(0, d, step=_L)
            def _(c0):
                v = comb[r, pl.ds(c0, _L)]
                comb[r, pl.ds(d + c0, _L)] = v * v

        plsc.subcore_barrier()

        adds = []
        for g in range(ngrp):
            adds.append(pltpu.async_copy(comb.at[pl.ds(g * _GRP, _GRP), :],
                                         acc.at[idx_v.at[g]], add_sem,
                                         add=True))
        adds.append(pltpu.async_copy(hist, acc_c.at[iota_v], add_sem,
                                     add=True))
        for a in adds:
            a.wait()

        plsc.subcore_barrier()

        obase = cid * _VOCAB_PAD + vbase
        w0 = pltpu.async_copy(acc.at[pl.ds(vbase, rows_w), :],
                              acc_hbm.at[pl.ds(obase, rows_w), :], add_sem)

        @pl.when(sid == 0)
        def _():
            pltpu.async_copy(acc_c,
                             cnt_hbm.at[pl.ds(cid * _HR, _HR), :],
                             add_sem).wait()

        w0.wait()

    return k(x, t)


def _finalize_tc(acc, counts):
    vp = _VOCAB_PAD

    def body(a_ref, c_ref, loss_ref, nrep_ref):
        s = a_ref[:vp, 0:64] + a_ref[vp:, 0:64]
        q = a_ref[:vp, 64:128] + a_ref[vp:, 64:128]
        c8 = c_ref[0:8, :] + c_ref[_HR:_HR + 8, :]
        i_row = lax.broadcasted_iota(jnp.int32, (vp, 1), 0)
        hi = lax.shift_right_logical(i_row, 7)
        lo = lax.bitwise_and(i_row, 127)
        lane = lax.broadcasted_iota(jnp.int32, (vp, _W), 1)
        sel = jnp.zeros((vp, _W), jnp.float32)
        for r in range(8):
            sel = jnp.where(hi == r, c8[r:r + 1, :], sel)
        c = jnp.sum(jnp.where(lane == lo, sel, 0.0), axis=1, keepdims=True)
        mean = s / jnp.maximum(c, 1.0)
        ss = q - c * mean * mean
        var = ss / jnp.maximum(c - 1.0, 1.0)
        var_mean = jnp.sum(var, axis=1, keepdims=True) / var.shape[1]
        repeated = c >= float(_MIN_OCC)
        nrep = jnp.sum(repeated.astype(jnp.int32))
        total = jnp.sum(jnp.where(repeated, var_mean, 0.0))
        avg = total / jnp.maximum(nrep.astype(jnp.float32), 1.0)
        loss = jnp.clip(1.0 - avg, 0.0, None)
        loss_ref[0, 0] = jnp.where(nrep > 0, loss, jnp.float32(0.0))
        nrep_ref[0, 0] = nrep

    return pl.pallas_call(
        body,
        out_shape=(
            jax.ShapeDtypeStruct((1, 1), jnp.float32),
            jax.ShapeDtypeStruct((1, 1), jnp.int32),
        ),
        out_specs=(
            pl.BlockSpec(memory_space=pltpu.SMEM),
            pl.BlockSpec(memory_space=pltpu.SMEM),
        ),
    )(acc, counts)


@jax.jit
def kernel(semantic_state, token_ids):
    b, t_len, d = semantic_state.shape
    n = b * t_len
    x = semantic_state.reshape(n, d)
    nthi = t_len // _GRP
    t = token_ids.astype(jnp.int32).reshape(b, nthi, _GRP).transpose(1, 0, 2)
    acc, counts = _sc_segment_stats(x, t)
    loss, nrep = _finalize_tc(acc, counts)
    return loss[0, 0], nrep[0, 0]
